# baseline (speedup 1.0000x reference)
import jax
import jax.numpy as jnp
from jax.experimental import pallas as pl
from jax.experimental.pallas import tpu as pltpu

M, K, N = 1024, 100000, 16
BK = 4096
NK = (K + BK - 1) // BK
REM = K - (NK - 1) * BK


def _mm_kernel(x_ref, w_ref, o_ref):
    k = pl.program_id(0)

    @pl.when(k == 0)
    def _():
        o_ref[...] = jnp.zeros_like(o_ref)

    @pl.when(k < NK - 1)
    def _():
        o_ref[...] += jnp.dot(x_ref[...], w_ref[...],
                              preferred_element_type=jnp.float32)

    @pl.when(k == NK - 1)
    def _():
        mask = jax.lax.broadcasted_iota(jnp.int32, (M, BK), 1) < REM
        xm = jnp.where(mask, x_ref[...], 0.0)
        o_ref[...] += jnp.dot(xm, w_ref[...],
                              preferred_element_type=jnp.float32)


def kernel(x, W):
    return pl.pallas_call(
        _mm_kernel,
        grid=(NK,),
        in_specs=[
            pl.BlockSpec((M, BK), lambda k: (0, k)),
            pl.BlockSpec((BK, N), lambda k: (k, 0)),
        ],
        out_specs=pl.BlockSpec((M, N), lambda k: (0, 0)),
        out_shape=jax.ShapeDtypeStruct((M, N), jnp.float32),
        compiler_params=pltpu.CompilerParams(
            dimension_semantics=("arbitrary",)),
    )(x, W)


# speedup vs baseline: 1.0068x; 1.0068x over previous
import jax
import jax.numpy as jnp
from jax.experimental import pallas as pl
from jax.experimental.pallas import tpu as pltpu

M, K, N = 1024, 100000, 16
BK = 4096
NK = (K + BK - 1) // BK
REM = K - (NK - 1) * BK
NS = 4
SM = M // NS


def _mm_kernel(x0_ref, x1_ref, x2_ref, x3_ref, w_ref, o_ref):
    k = pl.program_id(0)

    @pl.when(k == 0)
    def _():
        o_ref[...] = jnp.zeros_like(o_ref)

    w = w_ref[...]

    @pl.when(k < NK - 1)
    def _():
        for i, xr in enumerate((x0_ref, x1_ref, x2_ref, x3_ref)):
            o_ref[i * SM:(i + 1) * SM, :] += jnp.dot(
                xr[...], w, preferred_element_type=jnp.float32)

    @pl.when(k == NK - 1)
    def _():
        mask = jax.lax.broadcasted_iota(jnp.int32, (SM, BK), 1) < REM
        for i, xr in enumerate((x0_ref, x1_ref, x2_ref, x3_ref)):
            xm = jnp.where(mask, xr[...], 0.0)
            o_ref[i * SM:(i + 1) * SM, :] += jnp.dot(
                xm, w, preferred_element_type=jnp.float32)


def kernel(x, W):
    def stripe_spec(i):
        return pl.BlockSpec((SM, BK), lambda k, i=i: (i, k))

    return pl.pallas_call(
        _mm_kernel,
        grid=(NK,),
        in_specs=[stripe_spec(i) for i in range(NS)] + [
            pl.BlockSpec((BK, N), lambda k: (k, 0)),
        ],
        out_specs=pl.BlockSpec((M, N), lambda k: (0, 0)),
        out_shape=jax.ShapeDtypeStruct((M, N), jnp.float32),
        compiler_params=pltpu.CompilerParams(
            dimension_semantics=("arbitrary",)),
    )(x, x, x, x, W)
